# Initial kernel scaffold; baseline (speedup 1.0000x reference)
import jax
import jax.numpy as jnp
from jax.experimental import pallas as pl


def kernel(x, edge_index, W_proj, b_proj, W_layers, b_layers, W_p1, b_p1, W_p2, b_p2):
    raise NotImplementedError("write your pallas kernel here")


# speedup vs baseline: 6.3527x; 6.3527x over previous
import functools

import jax
import jax.numpy as jnp
from jax import lax
from jax.experimental import pallas as pl
from jax.experimental.pallas import tpu as pltpu
from jax.experimental.pallas import tpu_sc as plsc

N_NODES = 10000
N_EDGES = 320000
DIM = 128
OUT_DIM = 256
NUM_LAYERS = 3

NCORES = 2
NSUB = 16
NTILES = NCORES * NSUB
CHUNK = 128
NCHUNKS = N_EDGES // CHUNK
CH_PER_CORE = NCHUNKS // NCORES
COPY_ROWS = 624
COPY_REM = N_NODES - COPY_ROWS * NSUB
ZROWS = 104

_f32 = jnp.float32


def _zero_vmem2(ref, rows, cols):
    zeros16 = jnp.zeros((16,), _f32)
    per_row = cols // 16

    def body(i, _):
        r = i // per_row
        col = (i % per_row) * 16
        ref[r, pl.ds(col, 16)] = zeros16
        return 0

    lax.fori_loop(0, rows * per_row, body, 0)


def _agg_body(compute_deg, *args):
    if compute_deg:
        (h_hbm, src_hbm, dst_hbm, part_hbm, deg_hbm,
         acc_sh, src_v, dst_v, rows_v, zbuf, deg_sh, ones_v, sem) = args
    else:
        (h_hbm, src_hbm, dst_hbm, part_hbm,
         acc_sh, src_v, dst_v, rows_v, zbuf, sem) = args

    c = lax.axis_index("c")
    s = lax.axis_index("s")

    _zero_vmem2(zbuf, ZROWS, DIM)
    row0 = s * COPY_ROWS
    for k in range(COPY_ROWS // ZROWS):
        pltpu.sync_copy(zbuf, acc_sh.at[pl.ds(row0 + k * ZROWS, ZROWS)])

    @pl.when(s == NSUB - 1)
    def _():
        pltpu.sync_copy(zbuf.at[pl.ds(0, COPY_REM)],
                        acc_sh.at[pl.ds(NSUB * COPY_ROWS, COPY_REM)])

    if compute_deg:
        ones16 = jnp.full((16,), 1.0, _f32)
        for i in range(CHUNK // 16):
            ones_v[pl.ds(i * 16, 16)] = ones16
        _zero_vmem2(zbuf, 1, DIM)
        zr = zbuf.at[0]
        for k in range(5):
            pltpu.sync_copy(zr, deg_sh.at[pl.ds(s * COPY_ROWS + k * DIM, DIM)])

    plsc.subcore_barrier()

    base = c * CH_PER_CORE + s
    nch = (CH_PER_CORE - s + NSUB - 1) // NSUB

    def body(i, _):
        ch = base + i * NSUB
        pltpu.sync_copy(src_hbm.at[ch, 0], src_v)
        pltpu.sync_copy(dst_hbm.at[ch, 0], dst_v)
        pltpu.async_copy(h_hbm.at[src_v], rows_v, sem).wait()
        pltpu.sync_copy(rows_v, acc_sh.at[dst_v], add=True)
        if compute_deg:
            pltpu.sync_copy(ones_v, deg_sh.at[dst_v], add=True)
        return 0

    lax.fori_loop(0, nch, body, 0)

    plsc.subcore_barrier()

    pltpu.sync_copy(acc_sh.at[pl.ds(row0, COPY_ROWS)],
                    part_hbm.at[c, pl.ds(row0, COPY_ROWS)])

    @pl.when(s == NSUB - 1)
    def _():
        pltpu.sync_copy(acc_sh.at[pl.ds(NSUB * COPY_ROWS, COPY_REM)],
                        part_hbm.at[c, pl.ds(NSUB * COPY_ROWS, COPY_REM)])

    if compute_deg:
        @pl.when(s == 0)
        def _():
            pltpu.sync_copy(deg_sh, deg_hbm.at[c, 0])


def _make_agg(compute_deg):
    mesh = plsc.VectorSubcoreMesh(core_axis_name="c", subcore_axis_name="s",
                                  num_cores=NCORES, num_subcores=NSUB)
    outs = [jax.ShapeDtypeStruct((NCORES, N_NODES, DIM), _f32)]
    scratch = [
        pltpu.VMEM_SHARED((N_NODES, DIM), _f32),
        pltpu.VMEM((CHUNK,), jnp.int32),
        pltpu.VMEM((CHUNK,), jnp.int32),
        pltpu.VMEM((CHUNK, DIM), _f32),
        pltpu.VMEM((ZROWS, DIM), _f32),
    ]
    if compute_deg:
        outs.append(jax.ShapeDtypeStruct((NCORES, 1, N_NODES), _f32))
        scratch.append(pltpu.VMEM_SHARED((N_NODES,), _f32))
        scratch.append(pltpu.VMEM((CHUNK,), _f32))
    scratch.append(pltpu.SemaphoreType.DMA)
    return pl.kernel(
        functools.partial(_agg_body, compute_deg),
        out_type=tuple(outs) if compute_deg else outs[0],
        mesh=mesh,
        scratch_types=scratch,
    )


_agg_deg = _make_agg(True)
_agg = _make_agg(False)


def _proj_body(x_ref, w_ref, b_ref, o_ref):
    o_ref[...] = (jnp.dot(x_ref[...], w_ref[...],
                          preferred_element_type=_f32) + b_ref[...])


_proj = pl.pallas_call(
    _proj_body,
    grid=(10,),
    in_specs=[
        pl.BlockSpec((N_NODES // 10, DIM), lambda i: (i, 0)),
        pl.BlockSpec((DIM, DIM), lambda i: (0, 0)),
        pl.BlockSpec((1, DIM), lambda i: (0, 0)),
    ],
    out_specs=pl.BlockSpec((N_NODES // 10, DIM), lambda i: (i, 0)),
    out_shape=jax.ShapeDtypeStruct((N_NODES, DIM), _f32),
)


def _invdeg_body(d_ref, o_ref):
    deg = jnp.sum(d_ref[...], axis=0)
    o_ref[...] = (1.0 / jnp.maximum(deg, 1.0))[:, None]


_invdeg = pl.pallas_call(
    _invdeg_body,
    grid=(1,),
    in_specs=[pl.BlockSpec((NCORES, N_NODES), lambda i: (0, 0))],
    out_specs=pl.BlockSpec((N_NODES, 1), lambda i: (0, 0)),
    out_shape=jax.ShapeDtypeStruct((N_NODES, 1), _f32),
)


def _update_body(h_ref, p0_ref, p1_ref, inv_ref, w_ref, b_ref, o_ref):
    msg = (p0_ref[...] + p1_ref[...]) * inv_ref[...]
    o_ref[...] = jnp.maximum(
        h_ref[...] + jnp.dot(msg, w_ref[...], preferred_element_type=_f32)
        + b_ref[...], 0.0)


_update = pl.pallas_call(
    _update_body,
    grid=(10,),
    in_specs=[
        pl.BlockSpec((N_NODES // 10, DIM), lambda i: (i, 0)),
        pl.BlockSpec((N_NODES // 10, DIM), lambda i: (i, 0)),
        pl.BlockSpec((N_NODES // 10, DIM), lambda i: (i, 0)),
        pl.BlockSpec((N_NODES // 10, 1), lambda i: (i, 0)),
        pl.BlockSpec((DIM, DIM), lambda i: (0, 0)),
        pl.BlockSpec((1, DIM), lambda i: (0, 0)),
    ],
    out_specs=pl.BlockSpec((N_NODES // 10, DIM), lambda i: (i, 0)),
    out_shape=jax.ShapeDtypeStruct((N_NODES, DIM), _f32),
)


def _head_body(h_ref, w1_ref, b1_ref, w2_ref, b2_ref, o_ref):
    g = jnp.sum(h_ref[...], axis=0, keepdims=True) * (1.0 / N_NODES)
    hid = jnp.maximum(
        jnp.dot(g, w1_ref[...], preferred_element_type=_f32) + b1_ref[...],
        0.0)
    o_ref[...] = (jnp.dot(hid, w2_ref[...], preferred_element_type=_f32)
                  + b2_ref[...]).reshape(OUT_DIM)


_head = pl.pallas_call(
    _head_body,
    grid=(1,),
    in_specs=[
        pl.BlockSpec((N_NODES, DIM), lambda i: (0, 0)),
        pl.BlockSpec((DIM, DIM), lambda i: (0, 0)),
        pl.BlockSpec((1, DIM), lambda i: (0, 0)),
        pl.BlockSpec((DIM, OUT_DIM), lambda i: (0, 0)),
        pl.BlockSpec((1, OUT_DIM), lambda i: (0, 0)),
    ],
    out_specs=pl.BlockSpec((OUT_DIM,), lambda i: (0,)),
    out_shape=jax.ShapeDtypeStruct((OUT_DIM,), _f32),
)


def kernel(x, edge_index, W_proj, b_proj, W_layers, b_layers, W_p1, b_p1,
           W_p2, b_p2):
    src = edge_index[0].astype(jnp.int32).reshape(NCHUNKS, 1, CHUNK)
    dst = edge_index[1].astype(jnp.int32).reshape(NCHUNKS, 1, CHUNK)

    h = _proj(x, W_proj, b_proj.reshape(1, DIM))

    part, deg32 = _agg_deg(h, src, dst)
    invdeg = _invdeg(deg32.reshape(NCORES, N_NODES))
    h = _update(h, part[0], part[1], invdeg, W_layers[0],
                b_layers[0].reshape(1, DIM))
    for l in range(1, NUM_LAYERS):
        part = _agg(h, src, dst)
        h = _update(h, part[0], part[1], invdeg, W_layers[l],
                    b_layers[l].reshape(1, DIM))

    return _head(h, W_p1, b_p1.reshape(1, DIM), W_p2, b_p2.reshape(1, OUT_DIM))
